# baseline (speedup 1.0000x reference)
import functools

import jax
import jax.numpy as jnp
from jax import lax
from jax.experimental import pallas as pl
from jax.experimental.pallas import tpu as pltpu
from jax.experimental.pallas import tpu_sc as plsc

D_MODEL = 1024
D_FF = 2048
N_EXPERTS = 8
MAX_K = 4
THRESHOLD = 0.8
T_TOKENS = 2048

NW = 32
TPW = T_TOKENS // NW
F_BLK = 512


def _lane():
    return lax.broadcasted_iota(jnp.int32, (16,), 0)


def _gates_from_logits(lv):
    m = lv[0]
    for e in range(1, N_EXPERTS):
        m = jnp.maximum(m, lv[e])
    ex = [jnp.exp(lv[e] - m) for e in range(N_EXPERTS)]
    ssum = ex[0]
    for e in range(1, N_EXPERTS):
        ssum = ssum + ex[e]
    p = [ex[e] / ssum for e in range(N_EXPERTS)]
    gts = []
    for j in range(N_EXPERTS):
        above = jnp.zeros((16,), jnp.float32)
        csb = jnp.zeros((16,), jnp.float32)
        for e in range(N_EXPERTS):
            if e == j:
                continue
            ind = (p[e] >= p[j]) if e < j else (p[e] > p[j])
            above = above + jnp.where(ind, 1.0, 0.0)
            csb = csb + jnp.where(ind, p[e], 0.0)
        mask = (csb < THRESHOLD) & (above < float(MAX_K))
        gts.append(jnp.where(mask, p[j], 0.0))
    return gts


def _sc_route(logits_hbm, gates_hbm, lg_v, gb_v):
    w = lax.axis_index("s") * 2 + lax.axis_index("c")
    lane = _lane()
    pltpu.sync_copy(logits_hbm.at[pl.ds(w * 512, 512)], lg_v)
    for c2 in range(4):
        base = c2 * 128
        lv = [plsc.load_gather(lg_v, [base + lane * 8 + e])
              for e in range(N_EXPERTS)]
        gts = _gates_from_logits(lv)
        for j in range(N_EXPERTS):
            plsc.store_scatter(gb_v, [base + lane * 8 + j], gts[j])
    pltpu.sync_copy(gb_v, gates_hbm.at[pl.ds(w * 512, 512)])


def _ffn_kernel(x_ref, g_ref, wg_ref, wu_ref, wd_ref, out_ref):
    e = pl.program_id(0)
    f = pl.program_id(1)

    x = x_ref[...]
    hg = jnp.dot(x, wg_ref[0], preferred_element_type=jnp.float32)
    hu = jnp.dot(x, wu_ref[0], preferred_element_type=jnp.float32)
    h = (hg * jax.nn.sigmoid(hg) * hu).astype(jnp.bfloat16)
    y = jnp.dot(h, wd_ref[0], preferred_element_type=jnp.float32)

    gates = g_ref[...]
    lane = jax.lax.broadcasted_iota(jnp.int32, gates.shape, 1)
    g = jnp.sum(jnp.where(lane == e, gates, 0.0), axis=-1, keepdims=True)
    contrib = y * g

    @pl.when((e == 0) & (f == 0))
    def _():
        out_ref[...] = contrib

    @pl.when((e != 0) | (f != 0))
    def _():
        out_ref[...] = out_ref[...] + contrib


@jax.jit
def kernel(x, W_router, W_gate, W_up, W_down):
    logits = (x @ W_router).reshape(-1)
    xb = x.astype(jnp.bfloat16)
    wg = W_gate.astype(jnp.bfloat16)
    wu = W_up.astype(jnp.bfloat16)
    wd = W_down.astype(jnp.bfloat16)

    route = pl.kernel(
        _sc_route,
        mesh=plsc.VectorSubcoreMesh(core_axis_name="c", subcore_axis_name="s"),
        out_type=jax.ShapeDtypeStruct((T_TOKENS * N_EXPERTS,), jnp.float32),
        scratch_types=[
            pltpu.VMEM((512,), jnp.float32),
            pltpu.VMEM((512,), jnp.float32),
        ],
        compiler_params=pltpu.CompilerParams(
            use_tc_tiling_on_sc=False, needs_layout_passes=False),
    )
    gates = route(logits).reshape(T_TOKENS, N_EXPERTS)

    out = pl.pallas_call(
        _ffn_kernel,
        grid=(N_EXPERTS, D_FF // F_BLK),
        in_specs=[
            pl.BlockSpec((T_TOKENS, D_MODEL), lambda e, f: (0, 0)),
            pl.BlockSpec((T_TOKENS, N_EXPERTS), lambda e, f: (0, 0)),
            pl.BlockSpec((1, D_MODEL, F_BLK), lambda e, f: (e, 0, f)),
            pl.BlockSpec((1, D_MODEL, F_BLK), lambda e, f: (e, 0, f)),
            pl.BlockSpec((1, F_BLK, D_MODEL), lambda e, f: (e, f, 0)),
        ],
        out_specs=pl.BlockSpec((T_TOKENS, D_MODEL), lambda e, f: (0, 0)),
        out_shape=jax.ShapeDtypeStruct((T_TOKENS, D_MODEL), jnp.float32),
        compiler_params=pltpu.CompilerParams(
            dimension_semantics=("arbitrary", "arbitrary"),
        ),
    )(xb, gates, wg, wu, wd)
    return out


# speedup vs baseline: 1.0250x; 1.0250x over previous
import functools

import jax
import jax.numpy as jnp
from jax import lax
from jax.experimental import pallas as pl
from jax.experimental.pallas import tpu as pltpu
from jax.experimental.pallas import tpu_sc as plsc

D_MODEL = 1024
D_FF = 2048
N_EXPERTS = 8
MAX_K = 4
THRESHOLD = 0.8
T_TOKENS = 2048

NW = 32
TPW = T_TOKENS // NW
F_BLK = 1024


def _lane():
    return lax.broadcasted_iota(jnp.int32, (16,), 0)


def _gates_from_logits(lv):
    m = lv[0]
    for e in range(1, N_EXPERTS):
        m = jnp.maximum(m, lv[e])
    ex = [jnp.exp(lv[e] - m) for e in range(N_EXPERTS)]
    ssum = ex[0]
    for e in range(1, N_EXPERTS):
        ssum = ssum + ex[e]
    p = [ex[e] / ssum for e in range(N_EXPERTS)]
    gts = []
    for j in range(N_EXPERTS):
        above = jnp.zeros((16,), jnp.float32)
        csb = jnp.zeros((16,), jnp.float32)
        for e in range(N_EXPERTS):
            if e == j:
                continue
            ind = (p[e] >= p[j]) if e < j else (p[e] > p[j])
            above = above + jnp.where(ind, 1.0, 0.0)
            csb = csb + jnp.where(ind, p[e], 0.0)
        mask = (csb < THRESHOLD) & (above < float(MAX_K))
        gts.append(jnp.where(mask, p[j], 0.0))
    return gts


def _sc_route(logits_hbm, gates_hbm, lg_v, gb_v):
    w = lax.axis_index("s") * 2 + lax.axis_index("c")
    lane = _lane()
    pltpu.sync_copy(logits_hbm.at[pl.ds(w * 512, 512)], lg_v)
    for c2 in range(4):
        base = c2 * 128
        lv = [plsc.load_gather(lg_v, [base + lane * 8 + e])
              for e in range(N_EXPERTS)]
        gts = _gates_from_logits(lv)
        for j in range(N_EXPERTS):
            plsc.store_scatter(gb_v, [base + lane * 8 + j], gts[j])
    pltpu.sync_copy(gb_v, gates_hbm.at[pl.ds(w * 512, 512)])


def _ffn_kernel(x_ref, g_ref, wg_ref, wu_ref, wd_ref, out_ref):
    e = pl.program_id(0)
    f = pl.program_id(1)

    x = x_ref[...]
    half = F_BLK // 2
    y = None
    for c in range(2):
        wgc = wg_ref[0, :, pl.ds(c * half, half)]
        wuc = wu_ref[0, :, pl.ds(c * half, half)]
        wdc = wd_ref[0, pl.ds(c * half, half), :]
        hg = jnp.dot(x, wgc, preferred_element_type=jnp.float32)
        hu = jnp.dot(x, wuc, preferred_element_type=jnp.float32)
        h = (hg * jax.nn.sigmoid(hg) * hu).astype(jnp.bfloat16)
        yc = jnp.dot(h, wdc, preferred_element_type=jnp.float32)
        y = yc if y is None else y + yc

    gates = g_ref[...]
    lane = jax.lax.broadcasted_iota(jnp.int32, gates.shape, 1)
    g = jnp.sum(jnp.where(lane == e, gates, 0.0), axis=-1, keepdims=True)
    contrib = y * g

    @pl.when((e == 0) & (f == 0))
    def _():
        out_ref[...] = contrib

    @pl.when((e != 0) | (f != 0))
    def _():
        out_ref[...] = out_ref[...] + contrib


@jax.jit
def kernel(x, W_router, W_gate, W_up, W_down):
    logits = (x @ W_router).reshape(-1)
    xb = x.astype(jnp.bfloat16)
    wg = W_gate.astype(jnp.bfloat16)
    wu = W_up.astype(jnp.bfloat16)
    wd = W_down.astype(jnp.bfloat16)

    route = pl.kernel(
        _sc_route,
        mesh=plsc.VectorSubcoreMesh(core_axis_name="c", subcore_axis_name="s"),
        out_type=jax.ShapeDtypeStruct((T_TOKENS * N_EXPERTS,), jnp.float32),
        scratch_types=[
            pltpu.VMEM((512,), jnp.float32),
            pltpu.VMEM((512,), jnp.float32),
        ],
        compiler_params=pltpu.CompilerParams(
            use_tc_tiling_on_sc=False, needs_layout_passes=False),
    )
    gates = route(logits).reshape(T_TOKENS, N_EXPERTS)

    out = pl.pallas_call(
        _ffn_kernel,
        grid=(N_EXPERTS, D_FF // F_BLK),
        in_specs=[
            pl.BlockSpec((T_TOKENS, D_MODEL), lambda e, f: (0, 0)),
            pl.BlockSpec((T_TOKENS, N_EXPERTS), lambda e, f: (0, 0)),
            pl.BlockSpec((1, D_MODEL, F_BLK), lambda e, f: (e, 0, f)),
            pl.BlockSpec((1, D_MODEL, F_BLK), lambda e, f: (e, 0, f)),
            pl.BlockSpec((1, F_BLK, D_MODEL), lambda e, f: (e, f, 0)),
        ],
        out_specs=pl.BlockSpec((T_TOKENS, D_MODEL), lambda e, f: (0, 0)),
        out_shape=jax.ShapeDtypeStruct((T_TOKENS, D_MODEL), jnp.float32),
        compiler_params=pltpu.CompilerParams(
            dimension_semantics=("arbitrary", "arbitrary"),
        ),
    )(xb, gates, wg, wu, wd)
    return out


# speedup vs baseline: 1.0610x; 1.0351x over previous
import functools

import jax
import jax.numpy as jnp
from jax import lax
from jax.experimental import pallas as pl
from jax.experimental.pallas import tpu as pltpu
from jax.experimental.pallas import tpu_sc as plsc

D_MODEL = 1024
D_FF = 2048
N_EXPERTS = 8
MAX_K = 4
THRESHOLD = 0.8
T_TOKENS = 2048

NW = 32
TPW = T_TOKENS // NW
F_BLK = 1024


def _lane():
    return lax.broadcasted_iota(jnp.int32, (16,), 0)


def _gates_from_logits(lv):
    m = lv[0]
    for e in range(1, N_EXPERTS):
        m = jnp.maximum(m, lv[e])
    ex = [jnp.exp(lv[e] - m) for e in range(N_EXPERTS)]
    ssum = ex[0]
    for e in range(1, N_EXPERTS):
        ssum = ssum + ex[e]
    p = [ex[e] / ssum for e in range(N_EXPERTS)]
    gts = []
    for j in range(N_EXPERTS):
        above = jnp.zeros((16,), jnp.float32)
        csb = jnp.zeros((16,), jnp.float32)
        for e in range(N_EXPERTS):
            if e == j:
                continue
            ind = (p[e] >= p[j]) if e < j else (p[e] > p[j])
            above = above + jnp.where(ind, 1.0, 0.0)
            csb = csb + jnp.where(ind, p[e], 0.0)
        mask = (csb < THRESHOLD) & (above < float(MAX_K))
        gts.append(jnp.where(mask, p[j], 0.0))
    return gts


def _sc_route(logits_hbm, gates_hbm, lg_v, gb_v):
    w = lax.axis_index("s") * 2 + lax.axis_index("c")
    lane = _lane()
    pltpu.sync_copy(logits_hbm.at[pl.ds(w * 512, 512)], lg_v)
    for c2 in range(4):
        base = c2 * 128
        lv = [plsc.load_gather(lg_v, [base + lane * 8 + e])
              for e in range(N_EXPERTS)]
        gts = _gates_from_logits(lv)
        for j in range(N_EXPERTS):
            plsc.store_scatter(gb_v, [base + lane * 8 + j], gts[j])
    pltpu.sync_copy(gb_v, gates_hbm.at[pl.ds(w * 512, 512)])


def _ffn_kernel(x_ref, g_ref, wg_ref, wu_ref, wd_ref, out_ref):
    e = pl.program_id(0)

    x = x_ref[...]
    half = 256
    y = None
    for c in range(D_FF // 256):
        wgc = wg_ref[0, :, pl.ds(c * half, half)]
        wuc = wu_ref[0, :, pl.ds(c * half, half)]
        wdc = wd_ref[0, pl.ds(c * half, half), :]
        hg = jnp.dot(x, wgc, preferred_element_type=jnp.float32)
        hu = jnp.dot(x, wuc, preferred_element_type=jnp.float32)
        h = (hg * jax.nn.sigmoid(hg) * hu).astype(jnp.bfloat16)
        yc = jnp.dot(h, wdc, preferred_element_type=jnp.float32)
        y = yc if y is None else y + yc

    gates = g_ref[...]
    lane = jax.lax.broadcasted_iota(jnp.int32, gates.shape, 1)
    g = jnp.sum(jnp.where(lane == e, gates, 0.0), axis=-1, keepdims=True)
    contrib = y * g

    @pl.when(e == 0)
    def _():
        out_ref[...] = contrib

    @pl.when(e != 0)
    def _():
        out_ref[...] = out_ref[...] + contrib


@jax.jit
def kernel(x, W_router, W_gate, W_up, W_down):
    logits = (x @ W_router).reshape(-1)
    xb = x.astype(jnp.bfloat16)
    wg = W_gate.astype(jnp.bfloat16)
    wu = W_up.astype(jnp.bfloat16)
    wd = W_down.astype(jnp.bfloat16)

    route = pl.kernel(
        _sc_route,
        mesh=plsc.VectorSubcoreMesh(core_axis_name="c", subcore_axis_name="s"),
        out_type=jax.ShapeDtypeStruct((T_TOKENS * N_EXPERTS,), jnp.float32),
        scratch_types=[
            pltpu.VMEM((512,), jnp.float32),
            pltpu.VMEM((512,), jnp.float32),
        ],
        compiler_params=pltpu.CompilerParams(
            use_tc_tiling_on_sc=False, needs_layout_passes=False),
    )
    gates = route(logits).reshape(T_TOKENS, N_EXPERTS)

    out = pl.pallas_call(
        _ffn_kernel,
        grid=(N_EXPERTS,),
        in_specs=[
            pl.BlockSpec((T_TOKENS, D_MODEL), lambda e: (0, 0)),
            pl.BlockSpec((T_TOKENS, N_EXPERTS), lambda e: (0, 0)),
            pl.BlockSpec((1, D_MODEL, D_FF), lambda e: (e, 0, 0)),
            pl.BlockSpec((1, D_MODEL, D_FF), lambda e: (e, 0, 0)),
            pl.BlockSpec((1, D_FF, D_MODEL), lambda e: (e, 0, 0)),
        ],
        out_specs=pl.BlockSpec((T_TOKENS, D_MODEL), lambda e: (0, 0)),
        out_shape=jax.ShapeDtypeStruct((T_TOKENS, D_MODEL), jnp.float32),
        compiler_params=pltpu.CompilerParams(
            dimension_semantics=("arbitrary",),
        ),
    )(xb, gates, wg, wu, wd)
    return out
